# baseline (speedup 1.0000x reference)
import functools

import jax
import jax.numpy as jnp
from jax import lax
from jax.experimental import pallas as pl
from jax.experimental.pallas import tpu as pltpu
from jax.experimental.pallas import tpu_sc as plsc

N = 10000
E = 320000
D = 128
NP = 10240
NW = 32
EPW = E // NW
CW = 128
CH = (EPW + CW - 1) // CW
EPP = CH * CW
DA = 144
ROWS_PER_TILE = NP // 16


def _mm_body(x_ref, ws_ref, wl_ref, bl_ref, wd_ref, attd_ref, atts_ref,
             h_ref, skip_ref, as_ref, ad_ref, mas_ref, mad_ref):
    i = pl.program_id(0)
    xb = x_ref[...]
    h = jnp.dot(xb, ws_ref[...], preferred_element_type=jnp.float32)
    h_ref[:, :D] = h
    lane = jax.lax.broadcasted_iota(jnp.int32, (xb.shape[0], DA - D), 1)
    h_ref[:, D:] = jnp.where(lane == 0, 1.0, 0.0)
    skip_ref[...] = (jnp.dot(xb, wl_ref[...], preferred_element_type=jnp.float32)
                     + bl_ref[...][None, :])
    a_s = jnp.sum(h * atts_ref[...][None, :], axis=1)
    as_ref[...] = a_s
    wdv = jnp.sum(wd_ref[...] * attd_ref[...][None, :], axis=1)
    a_d = jnp.sum(xb * wdv[None, :], axis=1)
    ad_ref[...] = a_d

    @pl.when(i == 0)
    def _():
        mas_ref[0, 0] = -jnp.inf
        mad_ref[0, 0] = -jnp.inf

    mas_ref[0, 0] = jnp.maximum(mas_ref[0, 0], jnp.max(a_s))
    mad_ref[0, 0] = jnp.maximum(mad_ref[0, 0], jnp.max(a_d))


def _mm(x, w_src, wl, bl, w_dst, att_dst, att_src):
    blk = 512
    grid = NP // blk
    return pl.pallas_call(
        _mm_body,
        grid=(grid,),
        in_specs=[
            pl.BlockSpec((blk, D), lambda i: (i, 0)),
            pl.BlockSpec((D, D), lambda i: (0, 0)),
            pl.BlockSpec((D, D), lambda i: (0, 0)),
            pl.BlockSpec((D,), lambda i: (0,)),
            pl.BlockSpec((D, D), lambda i: (0, 0)),
            pl.BlockSpec((D,), lambda i: (0,)),
            pl.BlockSpec((D,), lambda i: (0,)),
        ],
        out_specs=[
            pl.BlockSpec((blk, DA), lambda i: (i, 0)),
            pl.BlockSpec((blk, D), lambda i: (i, 0)),
            pl.BlockSpec((blk,), lambda i: (i,)),
            pl.BlockSpec((blk,), lambda i: (i,)),
            pl.BlockSpec((1, 1), lambda i: (0, 0), memory_space=pltpu.SMEM),
            pl.BlockSpec((1, 1), lambda i: (0, 0), memory_space=pltpu.SMEM),
        ],
        out_shape=[
            jax.ShapeDtypeStruct((NP, DA), jnp.float32),
            jax.ShapeDtypeStruct((NP, D), jnp.float32),
            jax.ShapeDtypeStruct((NP,), jnp.float32),
            jax.ShapeDtypeStruct((NP,), jnp.float32),
            jax.ShapeDtypeStruct((1, 1), jnp.float32),
            jax.ShapeDtypeStruct((1, 1), jnp.float32),
        ],
    )(x, w_src, wl, bl, w_dst, att_dst, att_src)


NI = 3


def _sc_edge_body(h_hbm, as_hbm, ad_hbm, idx_hbm, c_hbm, zr_hbm,
                  acc_out,
                  acc_sh, idxv, asg, adg, pbuf, rows, cv, gsem, isem):
    ci = lax.axis_index("c")
    si = lax.axis_index("s")
    wid = ci * 16 + si

    pltpu.sync_copy(zr_hbm, acc_sh.at[pl.ds(si * ROWS_PER_TILE, ROWS_PER_TILE)])
    pltpu.sync_copy(c_hbm, cv)
    plsc.subcore_barrier()

    cvec = cv[...]

    def fetch_idx(j, slot):
        pltpu.async_copy(idx_hbm.at[wid, j], idxv.at[slot], isem.at[slot])

    def wait_idx(j, slot):
        pltpu.make_async_copy(idx_hbm.at[wid, j], idxv.at[slot],
                              isem.at[slot]).wait()

    def launch_gathers(slot, buf):
        pltpu.async_copy(h_hbm.at[idxv.at[slot, 0]], rows.at[buf], gsem.at[buf])
        pltpu.async_copy(as_hbm.at[idxv.at[slot, 0]], asg.at[buf], gsem.at[buf])
        pltpu.async_copy(ad_hbm.at[idxv.at[slot, 1]], adg.at[buf], gsem.at[buf])

    def wait_gathers(slot, buf):
        pltpu.make_async_copy(h_hbm.at[idxv.at[slot, 0]], rows.at[buf],
                              gsem.at[buf]).wait()
        pltpu.make_async_copy(as_hbm.at[idxv.at[slot, 0]], asg.at[buf],
                              gsem.at[buf]).wait()
        pltpu.make_async_copy(ad_hbm.at[idxv.at[slot, 1]], adg.at[buf],
                              gsem.at[buf]).wait()

    fetch_idx(0, 0)
    fetch_idx(1, 1)
    wait_idx(0, 0)
    launch_gathers(0, 0)

    def chunk_body(i, carry):
        par = lax.rem(i, 2)
        nxt = lax.rem(i + 1, 2)
        slot = lax.rem(i, NI)
        nslot = lax.rem(i + 1, NI)
        fslot = lax.rem(i + 2, NI)

        @pl.when(i < CH - 1)
        def _():
            wait_idx(i + 1, nslot)
            launch_gathers(nslot, nxt)

            @pl.when(i < CH - 2)
            def _():
                fetch_idx(i + 2, fslot)

        wait_gathers(slot, par)

        def vreg_body(k, c2):
            e = asg[par, pl.ds(k * 16, 16)] + adg[par, pl.ds(k * 16, 16)]
            e = jnp.where(e > 0, e, 0.2 * e)
            pbuf[pl.ds(k * 16, 16)] = jnp.exp(e - cvec)
            return c2

        lax.fori_loop(0, CW // 16, vreg_body, 0, unroll=True)

        def row_body(g, c3):
            pv = pbuf[pl.ds(g * 16, 16)]
            for l in range(16):
                ps = pv[l]
                j = g * 16 + l
                for k2 in range(DA // 16):
                    rows[par, j, pl.ds(k2 * 16, 16)] = (
                        rows[par, j, pl.ds(k2 * 16, 16)] * ps)
            return c3

        lax.fori_loop(0, CW // 16, row_body, 0)

        pltpu.sync_copy(rows.at[par], acc_sh.at[idxv.at[slot, 1]], add=True)
        return carry

    lax.fori_loop(0, CH, chunk_body, 0)
    plsc.subcore_barrier()

    sl = pl.ds(si * ROWS_PER_TILE, ROWS_PER_TILE)
    pltpu.sync_copy(acc_sh.at[sl], acc_out.at[ci, sl])


@functools.partial(
    pl.kernel,
    out_type=jax.ShapeDtypeStruct((2, NP, DA), jnp.float32),
    mesh=plsc.VectorSubcoreMesh(core_axis_name="c", subcore_axis_name="s"),
    compiler_params=pltpu.CompilerParams(needs_layout_passes=False,
                                         use_tc_tiling_on_sc=False),
    scratch_types=[
        pltpu.VMEM_SHARED((NP, DA), jnp.float32),
        pltpu.VMEM((NI, 2, CW), jnp.int32),
        pltpu.VMEM((2, CW), jnp.float32),
        pltpu.VMEM((2, CW), jnp.float32),
        pltpu.VMEM((CW,), jnp.float32),
        pltpu.VMEM((2, CW, DA), jnp.float32),
        pltpu.VMEM((16,), jnp.float32),
        pltpu.SemaphoreType.DMA((2,)),
        pltpu.SemaphoreType.DMA((NI,)),
    ],
)
def _sc_edge(*refs):
    _sc_edge_body(*refs)


def _comb_body(acc_ref, skip_ref, b_ref, out_ref):
    full = acc_ref[0, :, :] + acc_ref[1, :, :]
    num = full[:, :D]
    dn = full[:, D:D + 1] + 1e-16
    h = num / dn + skip_ref[...] + b_ref[...][None, :]
    out_ref[...] = jnp.maximum(h, 0.0)


def _comb(acc, skip, b):
    blk = 512
    return pl.pallas_call(
        _comb_body,
        grid=(NP // blk,),
        in_specs=[
            pl.BlockSpec((2, blk, DA), lambda i: (0, i, 0)),
            pl.BlockSpec((blk, D), lambda i: (i, 0)),
            pl.BlockSpec((D,), lambda i: (0,)),
        ],
        out_specs=pl.BlockSpec((blk, D), lambda i: (i, 0)),
        out_shape=jax.ShapeDtypeStruct((NP, D), jnp.float32),
    )(acc, skip, b)


def _layer(x_pad, idx3, zr, w_src, w_dst, att_src, att_dst, b, wl, bl):
    h, skip, a_s, a_d, mas, mad = _mm(x_pad, w_src, wl, bl, w_dst, att_dst, att_src)
    cb = mas[0, 0] + mad[0, 0]
    c = jnp.where(cb > 0, cb, 0.2 * cb)
    cvec = jnp.full((16,), c, jnp.float32)
    acc = _sc_edge(h, a_s, a_d, idx3, cvec, zr)
    return _comb(acc, skip, b)


def kernel(x, edge_index, W1_src, W1_dst, att1_src, att1_dst, b1, Wl1, bl1,
           W2_src, W2_dst, att2_src, att2_dst, b2, Wl2, bl2):
    x_pad = jnp.pad(x, ((0, NP - N), (0, 0)))
    src = edge_index[0].astype(jnp.int32).reshape(NW, EPW)
    dst = edge_index[1].astype(jnp.int32).reshape(NW, EPW)
    pad = ((0, 0), (0, EPP - EPW))
    src3 = jnp.pad(src, pad, constant_values=NP - 1).reshape(NW, CH, CW)
    dst3 = jnp.pad(dst, pad, constant_values=NP - 1).reshape(NW, CH, CW)
    idx3 = jnp.stack([src3, dst3], axis=2)
    zr = jnp.zeros((ROWS_PER_TILE, DA), jnp.float32)

    h = _layer(x_pad, idx3, zr,
               W1_src, W1_dst, att1_src, att1_dst, b1, Wl1, bl1)
    out = _layer(h, idx3, zr,
                 W2_src, W2_dst, att2_src, att2_dst, b2, Wl2, bl2)
    return out[:N]


# speedup vs baseline: 1.0884x; 1.0884x over previous
import functools

import jax
import jax.numpy as jnp
from jax import lax
from jax.experimental import pallas as pl
from jax.experimental.pallas import tpu as pltpu
from jax.experimental.pallas import tpu_sc as plsc

N = 10000
E = 320000
D = 128
NP = 10240
NW = 32
EPW = E // NW
CW = 64
CH = (EPW + CW - 1) // CW
EPP = CH * CW
DA = 144
NACC = 10176
ACC_PER_TILE = NACC // 16
PAD_DST = 10100
ROWS_PER_TILE = NP // 16


def _mm_body(x_ref, ws_ref, wl_ref, bl_ref, wd_ref, attd_ref, atts_ref,
             h_ref, skip_ref, as_ref, ad_ref, mas_ref, mad_ref):
    i = pl.program_id(0)
    xb = x_ref[...]
    h = jnp.dot(xb, ws_ref[...], preferred_element_type=jnp.float32)
    h_ref[:, :D] = h
    lane = jax.lax.broadcasted_iota(jnp.int32, (xb.shape[0], DA - D), 1)
    h_ref[:, D:] = jnp.where(lane == 0, 1.0, 0.0)
    skip_ref[...] = (jnp.dot(xb, wl_ref[...], preferred_element_type=jnp.float32)
                     + bl_ref[...][None, :])
    a_s = jnp.sum(h * atts_ref[...][None, :], axis=1)
    as_ref[...] = a_s
    wdv = jnp.sum(wd_ref[...] * attd_ref[...][None, :], axis=1)
    a_d = jnp.sum(xb * wdv[None, :], axis=1)
    ad_ref[...] = a_d

    @pl.when(i == 0)
    def _():
        mas_ref[0, 0] = -jnp.inf
        mad_ref[0, 0] = -jnp.inf

    mas_ref[0, 0] = jnp.maximum(mas_ref[0, 0], jnp.max(a_s))
    mad_ref[0, 0] = jnp.maximum(mad_ref[0, 0], jnp.max(a_d))


def _mm(x, w_src, wl, bl, w_dst, att_dst, att_src):
    blk = 512
    grid = NP // blk
    return pl.pallas_call(
        _mm_body,
        grid=(grid,),
        in_specs=[
            pl.BlockSpec((blk, D), lambda i: (i, 0)),
            pl.BlockSpec((D, D), lambda i: (0, 0)),
            pl.BlockSpec((D, D), lambda i: (0, 0)),
            pl.BlockSpec((D,), lambda i: (0,)),
            pl.BlockSpec((D, D), lambda i: (0, 0)),
            pl.BlockSpec((D,), lambda i: (0,)),
            pl.BlockSpec((D,), lambda i: (0,)),
        ],
        out_specs=[
            pl.BlockSpec((blk, DA), lambda i: (i, 0)),
            pl.BlockSpec((blk, D), lambda i: (i, 0)),
            pl.BlockSpec((blk,), lambda i: (i,)),
            pl.BlockSpec((blk,), lambda i: (i,)),
            pl.BlockSpec((1, 1), lambda i: (0, 0), memory_space=pltpu.SMEM),
            pl.BlockSpec((1, 1), lambda i: (0, 0), memory_space=pltpu.SMEM),
        ],
        out_shape=[
            jax.ShapeDtypeStruct((NP, DA), jnp.float32),
            jax.ShapeDtypeStruct((NP, D), jnp.float32),
            jax.ShapeDtypeStruct((NP,), jnp.float32),
            jax.ShapeDtypeStruct((NP,), jnp.float32),
            jax.ShapeDtypeStruct((1, 1), jnp.float32),
            jax.ShapeDtypeStruct((1, 1), jnp.float32),
        ],
    )(x, w_src, wl, bl, w_dst, att_dst, att_src)


NI = 3


def _sc_edge_body(h_hbm, as_hbm, ad_hbm, idx_hbm, c_hbm, zr_hbm,
                  acc_out,
                  acc_sh, asv, adv, idxv, pbuf, rows, cv, gsem, ssem, isem):
    ci = lax.axis_index("c")
    si = lax.axis_index("s")
    wid = ci * 16 + si

    pltpu.sync_copy(zr_hbm, acc_sh.at[pl.ds(si * ACC_PER_TILE, ACC_PER_TILE)])
    pltpu.sync_copy(as_hbm, asv)
    pltpu.sync_copy(ad_hbm, adv)
    pltpu.sync_copy(c_hbm, cv)
    plsc.subcore_barrier()

    cvec = cv[...]

    def fetch_idx(j, slot):
        pltpu.async_copy(idx_hbm.at[wid, j], idxv.at[slot], isem.at[slot])

    def wait_idx(j, slot):
        pltpu.make_async_copy(idx_hbm.at[wid, j], idxv.at[slot],
                              isem.at[slot]).wait()

    fetch_idx(0, 0)
    fetch_idx(1, 1)
    wait_idx(0, 0)
    pltpu.async_copy(h_hbm.at[idxv.at[0, 0]], rows.at[0], gsem.at[0])

    def chunk_body(i, carry):
        par = lax.rem(i, 2)
        nxt = lax.rem(i + 1, 2)
        slot = lax.rem(i, NI)
        nslot = lax.rem(i + 1, NI)
        fslot = lax.rem(i + 2, NI)
        pslot = lax.rem(i + NI - 1, NI)

        @pl.when(i < CH - 1)
        def _():
            @pl.when(i >= 1)
            def _():
                pltpu.make_async_copy(rows.at[nxt], acc_sh.at[idxv.at[pslot, 1]],
                                      ssem.at[nxt]).wait()

            wait_idx(i + 1, nslot)
            pltpu.async_copy(h_hbm.at[idxv.at[nslot, 0]], rows.at[nxt],
                             gsem.at[nxt])

        pltpu.make_async_copy(h_hbm.at[idxv.at[slot, 0]], rows.at[par],
                              gsem.at[par]).wait()

        def vreg_body(k, c2):
            sidx = idxv[slot, 0, pl.ds(k * 16, 16)]
            didx = idxv[slot, 1, pl.ds(k * 16, 16)]
            e = plsc.load_gather(asv, [sidx]) + plsc.load_gather(adv, [didx])
            e = jnp.where(e > 0, e, 0.2 * e)
            pbuf[pl.ds(k * 16, 16)] = jnp.exp(e - cvec)
            return c2

        lax.fori_loop(0, CW // 16, vreg_body, 0, unroll=True)

        def row_body(g, c3):
            pv = pbuf[pl.ds(g * 16, 16)]
            for l in range(16):
                ps = pv[l]
                j = g * 16 + l
                for k2 in range(DA // 16):
                    rows[par, j, pl.ds(k2 * 16, 16)] = (
                        rows[par, j, pl.ds(k2 * 16, 16)] * ps)
            return c3

        lax.fori_loop(0, CW // 16, row_body, 0)

        pltpu.async_copy(rows.at[par], acc_sh.at[idxv.at[slot, 1]],
                         ssem.at[par], add=True)

        @pl.when(i < CH - 2)
        def _():
            fetch_idx(i + 2, fslot)

        return carry

    lax.fori_loop(0, CH, chunk_body, 0)

    lb, ls = (CH - 1) % 2, (CH - 1) % NI
    pltpu.make_async_copy(rows.at[lb], acc_sh.at[idxv.at[ls, 1]],
                          ssem.at[lb]).wait()
    plsc.subcore_barrier()

    sl = pl.ds(si * ACC_PER_TILE, ACC_PER_TILE)
    pltpu.sync_copy(acc_sh.at[sl], acc_out.at[ci, sl])


@functools.partial(
    pl.kernel,
    out_type=jax.ShapeDtypeStruct((2, NACC, DA), jnp.float32),
    mesh=plsc.VectorSubcoreMesh(core_axis_name="c", subcore_axis_name="s"),
    compiler_params=pltpu.CompilerParams(needs_layout_passes=False,
                                         use_tc_tiling_on_sc=False),
    scratch_types=[
        pltpu.VMEM_SHARED((NACC, DA), jnp.float32),
        pltpu.VMEM((NP,), jnp.float32),
        pltpu.VMEM((NP,), jnp.float32),
        pltpu.VMEM((NI, 2, CW), jnp.int32),
        pltpu.VMEM((CW,), jnp.float32),
        pltpu.VMEM((2, CW, DA), jnp.float32),
        pltpu.VMEM((16,), jnp.float32),
        pltpu.SemaphoreType.DMA((2,)),
        pltpu.SemaphoreType.DMA((2,)),
        pltpu.SemaphoreType.DMA((NI,)),
    ],
)
def _sc_edge(*refs):
    _sc_edge_body(*refs)


def _comb_body(acc_ref, skip_ref, b_ref, out_ref):
    full = acc_ref[0, :, :] + acc_ref[1, :, :]
    num = full[:, :D]
    dn = full[:, D:D + 1] + 1e-16
    h = num / dn + skip_ref[...] + b_ref[...][None, :]
    out_ref[...] = jnp.maximum(h, 0.0)


def _comb(acc, skip, b):
    blk = 1272
    return pl.pallas_call(
        _comb_body,
        grid=(NACC // blk,),
        in_specs=[
            pl.BlockSpec((2, blk, DA), lambda i: (0, i, 0)),
            pl.BlockSpec((blk, D), lambda i: (i, 0)),
            pl.BlockSpec((D,), lambda i: (0,)),
        ],
        out_specs=pl.BlockSpec((blk, D), lambda i: (i, 0)),
        out_shape=jax.ShapeDtypeStruct((NACC, D), jnp.float32),
    )(acc, skip, b)


def _layer(x_pad, idx3, zr, w_src, w_dst, att_src, att_dst, b, wl, bl):
    h, skip, a_s, a_d, mas, mad = _mm(x_pad, w_src, wl, bl, w_dst, att_dst, att_src)
    cb = mas[0, 0] + mad[0, 0]
    c = jnp.where(cb > 0, cb, 0.2 * cb)
    cvec = jnp.full((16,), c, jnp.float32)
    acc = _sc_edge(h, a_s, a_d, idx3, cvec, zr)
    out = _comb(acc, skip[:NACC], b)
    return jnp.pad(out, ((0, NP - NACC), (0, 0)))


def kernel(x, edge_index, W1_src, W1_dst, att1_src, att1_dst, b1, Wl1, bl1,
           W2_src, W2_dst, att2_src, att2_dst, b2, Wl2, bl2):
    x_pad = jnp.pad(x, ((0, NP - N), (0, 0)))
    src = edge_index[0].astype(jnp.int32).reshape(NW, EPW)
    dst = edge_index[1].astype(jnp.int32).reshape(NW, EPW)
    pad = ((0, 0), (0, EPP - EPW))
    src3 = jnp.pad(src, pad, constant_values=NP - 1).reshape(NW, CH, CW)
    dst3 = jnp.pad(dst, pad, constant_values=PAD_DST).reshape(NW, CH, CW)
    idx3 = jnp.stack([src3, dst3], axis=2)
    zr = jnp.zeros((ACC_PER_TILE, DA), jnp.float32)

    h = _layer(x_pad, idx3, zr,
               W1_src, W1_dst, att1_src, att1_dst, b1, Wl1, bl1)
    out = _layer(h, idx3, zr,
                 W2_src, W2_dst, att2_src, att2_dst, b2, Wl2, bl2)
    return out[:N]


# speedup vs baseline: 1.4446x; 1.3273x over previous
import functools

import jax
import jax.numpy as jnp
from jax import lax
from jax.experimental import pallas as pl
from jax.experimental.pallas import tpu as pltpu
from jax.experimental.pallas import tpu_sc as plsc

N = 10000
E = 320000
D = 128
NP = 10240
NW = 32
EPW = E // NW
CW = 128
CH = (EPW + CW - 1) // CW
EPP = CH * CW
DA = 144
NACC = 10160
ACC_PER_TILE = NACC // 16
PAD_DST = 10100
ROWS_PER_TILE = NP // 16


def _mm_body(x_ref, ws_ref, wl_ref, bl_ref, wd_ref, attd_ref, atts_ref,
             h_ref, skip_ref, as_ref, ad_ref, mas_ref, mad_ref):
    i = pl.program_id(0)
    xb = x_ref[...]
    h = jnp.dot(xb, ws_ref[...], preferred_element_type=jnp.float32)
    h_ref[:, :D] = h
    lane = jax.lax.broadcasted_iota(jnp.int32, (xb.shape[0], DA - D), 1)
    h_ref[:, D:] = jnp.where(lane == 0, 1.0, 0.0)
    skip_ref[...] = (jnp.dot(xb, wl_ref[...], preferred_element_type=jnp.float32)
                     + bl_ref[...][None, :])
    a_s = jnp.sum(h * atts_ref[...][None, :], axis=1)
    as_ref[...] = a_s
    wdv = jnp.sum(wd_ref[...] * attd_ref[...][None, :], axis=1)
    a_d = jnp.sum(xb * wdv[None, :], axis=1)
    ad_ref[...] = a_d

    @pl.when(i == 0)
    def _():
        mas_ref[0, 0] = -jnp.inf
        mad_ref[0, 0] = -jnp.inf

    mas_ref[0, 0] = jnp.maximum(mas_ref[0, 0], jnp.max(a_s))
    mad_ref[0, 0] = jnp.maximum(mad_ref[0, 0], jnp.max(a_d))


def _mm(x, w_src, wl, bl, w_dst, att_dst, att_src):
    blk = 512
    grid = NP // blk
    return pl.pallas_call(
        _mm_body,
        grid=(grid,),
        in_specs=[
            pl.BlockSpec((blk, D), lambda i: (i, 0)),
            pl.BlockSpec((D, D), lambda i: (0, 0)),
            pl.BlockSpec((D, D), lambda i: (0, 0)),
            pl.BlockSpec((D,), lambda i: (0,)),
            pl.BlockSpec((D, D), lambda i: (0, 0)),
            pl.BlockSpec((D,), lambda i: (0,)),
            pl.BlockSpec((D,), lambda i: (0,)),
        ],
        out_specs=[
            pl.BlockSpec((blk, DA), lambda i: (i, 0)),
            pl.BlockSpec((blk, D), lambda i: (i, 0)),
            pl.BlockSpec((blk,), lambda i: (i,)),
            pl.BlockSpec((blk,), lambda i: (i,)),
            pl.BlockSpec((1, 1), lambda i: (0, 0), memory_space=pltpu.SMEM),
            pl.BlockSpec((1, 1), lambda i: (0, 0), memory_space=pltpu.SMEM),
        ],
        out_shape=[
            jax.ShapeDtypeStruct((NP, DA), jnp.float32),
            jax.ShapeDtypeStruct((NP, D), jnp.float32),
            jax.ShapeDtypeStruct((NP,), jnp.float32),
            jax.ShapeDtypeStruct((NP,), jnp.float32),
            jax.ShapeDtypeStruct((1, 1), jnp.float32),
            jax.ShapeDtypeStruct((1, 1), jnp.float32),
        ],
    )(x, w_src, wl, bl, w_dst, att_dst, att_src)


NI = 3


def _sc_edge_body(h_hbm, as_hbm, ad_hbm, idx_hbm, c_hbm, zr_hbm,
                  acc_out,
                  acc_sh, asv, adv, idxv, pbuf, rows, cv, gsem):
    ci = lax.axis_index("c")
    si = lax.axis_index("s")
    wid = ci * 16 + si

    pltpu.sync_copy(zr_hbm, acc_sh.at[pl.ds(si * ACC_PER_TILE, ACC_PER_TILE)])
    pltpu.sync_copy(as_hbm, asv)
    pltpu.sync_copy(ad_hbm, adv)
    pltpu.sync_copy(c_hbm, cv)
    plsc.subcore_barrier()

    cvec = cv[...]

    def chunk_body(i, carry):
        pltpu.sync_copy(idx_hbm.at[wid, i], idxv)
        rcp = pltpu.async_copy(h_hbm.at[idxv.at[0]], rows, gsem)

        def vreg_body(k, c2):
            sidx = idxv[0, pl.ds(k * 16, 16)]
            didx = idxv[1, pl.ds(k * 16, 16)]
            e = plsc.load_gather(asv, [sidx]) + plsc.load_gather(adv, [didx])
            e = jnp.where(e > 0, e, 0.2 * e)
            pbuf[pl.ds(k * 16, 16)] = jnp.exp(e - cvec)
            return c2

        lax.fori_loop(0, CW // 16, vreg_body, 0, unroll=True)
        rcp.wait()

        def row_body(g, c3):
            pv = pbuf[pl.ds(g * 16, 16)]
            for l in range(16):
                ps = pv[l]
                j = g * 16 + l
                for k2 in range(DA // 16):
                    rows[j, pl.ds(k2 * 16, 16)] = (
                        rows[j, pl.ds(k2 * 16, 16)] * ps)
            return c3

        lax.fori_loop(0, CW // 16, row_body, 0)

        pltpu.sync_copy(rows, acc_sh.at[idxv.at[1]], add=True)
        return carry

    lax.fori_loop(0, CH, chunk_body, 0)
    plsc.subcore_barrier()

    sl = pl.ds(si * ACC_PER_TILE, ACC_PER_TILE)
    pltpu.sync_copy(acc_sh.at[sl], acc_out.at[ci, sl])


@functools.partial(
    pl.kernel,
    out_type=jax.ShapeDtypeStruct((2, NACC, DA), jnp.float32),
    mesh=plsc.VectorSubcoreMesh(core_axis_name="c", subcore_axis_name="s"),
    compiler_params=pltpu.CompilerParams(needs_layout_passes=False,
                                         use_tc_tiling_on_sc=False),
    scratch_types=[
        pltpu.VMEM_SHARED((NACC, DA), jnp.float32),
        pltpu.VMEM((NP,), jnp.float32),
        pltpu.VMEM((NP,), jnp.float32),
        pltpu.VMEM((2, CW), jnp.int32),
        pltpu.VMEM((CW,), jnp.float32),
        pltpu.VMEM((CW, DA), jnp.float32),
        pltpu.VMEM((16,), jnp.float32),
        pltpu.SemaphoreType.DMA,
    ],
)
def _sc_edge(*refs):
    _sc_edge_body(*refs)


def _comb_body(acc_ref, skip_ref, b_ref, out_ref):
    full = acc_ref[0, :, :] + acc_ref[1, :, :]
    num = full[:, :D]
    dn = full[:, D:D + 1] + 1e-16
    h = num / dn + skip_ref[...] + b_ref[...][None, :]
    out_ref[...] = jnp.maximum(h, 0.0)


def _comb(acc, skip, b):
    blk = 1272
    return pl.pallas_call(
        _comb_body,
        grid=(NACC // blk,),
        in_specs=[
            pl.BlockSpec((2, blk, DA), lambda i: (0, i, 0)),
            pl.BlockSpec((blk, D), lambda i: (i, 0)),
            pl.BlockSpec((D,), lambda i: (0,)),
        ],
        out_specs=pl.BlockSpec((blk, D), lambda i: (i, 0)),
        out_shape=jax.ShapeDtypeStruct((NACC, D), jnp.float32),
    )(acc, skip, b)


def _layer(x_pad, idx3, zr, w_src, w_dst, att_src, att_dst, b, wl, bl):
    h, skip, a_s, a_d, mas, mad = _mm(x_pad, w_src, wl, bl, w_dst, att_dst, att_src)
    cb = mas[0, 0] + mad[0, 0]
    c = jnp.where(cb > 0, cb, 0.2 * cb)
    cvec = jnp.full((16,), c, jnp.float32)
    acc = _sc_edge(h, a_s, a_d, idx3, cvec, zr)
    out = _comb(acc, skip[:NACC], b)
    return jnp.pad(out, ((0, NP - NACC), (0, 0)))


def kernel(x, edge_index, W1_src, W1_dst, att1_src, att1_dst, b1, Wl1, bl1,
           W2_src, W2_dst, att2_src, att2_dst, b2, Wl2, bl2):
    x_pad = jnp.pad(x, ((0, NP - N), (0, 0)))
    src = edge_index[0].astype(jnp.int32).reshape(NW, EPW)
    dst = edge_index[1].astype(jnp.int32).reshape(NW, EPW)
    pad = ((0, 0), (0, EPP - EPW))
    src3 = jnp.pad(src, pad, constant_values=NP - 1).reshape(NW, CH, CW)
    dst3 = jnp.pad(dst, pad, constant_values=PAD_DST).reshape(NW, CH, CW)
    idx3 = jnp.stack([src3, dst3], axis=2)
    zr = jnp.zeros((ACC_PER_TILE, DA), jnp.float32)

    h = _layer(x_pad, idx3, zr,
               W1_src, W1_dst, att1_src, att1_dst, b1, Wl1, bl1)
    out = _layer(h, idx3, zr,
                 W2_src, W2_dst, att2_src, att2_dst, b2, Wl2, bl2)
    return out[:N]


# speedup vs baseline: 1.4469x; 1.0016x over previous
import functools

import jax
import jax.numpy as jnp
from jax import lax
from jax.experimental import pallas as pl
from jax.experimental.pallas import tpu as pltpu
from jax.experimental.pallas import tpu_sc as plsc

N = 10000
E = 320000
D = 128
NP = 10240
NW = 32
EPW = E // NW
CW = 128
CH = (EPW + CW - 1) // CW
EPP = CH * CW
DA = 144
NACC = 10160
ACC_PER_TILE = NACC // 16
PAD_DST = 10100
ROWS_PER_TILE = NP // 16


def _mm_body(x_ref, ws_ref, wl_ref, bl_ref, wd_ref, attd_ref, atts_ref,
             h_ref, skip_ref, as_ref, ad_ref, mas_ref, mad_ref):
    i = pl.program_id(0)
    xb = x_ref[...]
    h = jnp.dot(xb, ws_ref[...], preferred_element_type=jnp.float32)
    h_ref[:, :D] = h
    lane = jax.lax.broadcasted_iota(jnp.int32, (xb.shape[0], DA - D), 1)
    h_ref[:, D:] = jnp.where(lane == 0, 1.0, 0.0)
    skip_ref[...] = (jnp.dot(xb, wl_ref[...], preferred_element_type=jnp.float32)
                     + bl_ref[...][None, :])
    a_s = jnp.sum(h * atts_ref[...][None, :], axis=1)
    as_ref[...] = a_s
    wdv = jnp.sum(wd_ref[...] * attd_ref[...][None, :], axis=1)
    a_d = jnp.sum(xb * wdv[None, :], axis=1)
    ad_ref[...] = a_d

    @pl.when(i == 0)
    def _():
        mas_ref[0, 0] = -jnp.inf
        mad_ref[0, 0] = -jnp.inf

    mas_ref[0, 0] = jnp.maximum(mas_ref[0, 0], jnp.max(a_s))
    mad_ref[0, 0] = jnp.maximum(mad_ref[0, 0], jnp.max(a_d))


def _mm(x, w_src, wl, bl, w_dst, att_dst, att_src):
    blk = 512
    grid = NP // blk
    return pl.pallas_call(
        _mm_body,
        grid=(grid,),
        in_specs=[
            pl.BlockSpec((blk, D), lambda i: (i, 0)),
            pl.BlockSpec((D, D), lambda i: (0, 0)),
            pl.BlockSpec((D, D), lambda i: (0, 0)),
            pl.BlockSpec((D,), lambda i: (0,)),
            pl.BlockSpec((D, D), lambda i: (0, 0)),
            pl.BlockSpec((D,), lambda i: (0,)),
            pl.BlockSpec((D,), lambda i: (0,)),
        ],
        out_specs=[
            pl.BlockSpec((blk, DA), lambda i: (i, 0)),
            pl.BlockSpec((blk, D), lambda i: (i, 0)),
            pl.BlockSpec((blk,), lambda i: (i,)),
            pl.BlockSpec((blk,), lambda i: (i,)),
            pl.BlockSpec((1, 1), lambda i: (0, 0), memory_space=pltpu.SMEM),
            pl.BlockSpec((1, 1), lambda i: (0, 0), memory_space=pltpu.SMEM),
        ],
        out_shape=[
            jax.ShapeDtypeStruct((NP, DA), jnp.float32),
            jax.ShapeDtypeStruct((NP, D), jnp.float32),
            jax.ShapeDtypeStruct((NP,), jnp.float32),
            jax.ShapeDtypeStruct((NP,), jnp.float32),
            jax.ShapeDtypeStruct((1, 1), jnp.float32),
            jax.ShapeDtypeStruct((1, 1), jnp.float32),
        ],
    )(x, w_src, wl, bl, w_dst, att_dst, att_src)


NI = 3


def _sc_edge_body(h_hbm, as_hbm, ad_hbm, idx_hbm, c_hbm, zr_hbm,
                  acc_out,
                  acc_sh, asv, adv, idxv, pbuf, rows, cv, gsem):
    ci = lax.axis_index("c")
    si = lax.axis_index("s")
    wid = ci * 16 + si

    pltpu.sync_copy(zr_hbm, acc_sh.at[pl.ds(si * ACC_PER_TILE, ACC_PER_TILE)])
    pltpu.sync_copy(as_hbm, asv)
    pltpu.sync_copy(ad_hbm, adv)
    pltpu.sync_copy(c_hbm, cv)
    plsc.subcore_barrier()

    cvec = cv[...]

    def chunk_body(i, carry):
        pltpu.sync_copy(idx_hbm.at[wid, i], idxv)
        rcp = pltpu.async_copy(h_hbm.at[idxv.at[0]], rows, gsem)

        def vreg_body(k, c2):
            sidx = idxv[0, pl.ds(k * 16, 16)]
            didx = idxv[1, pl.ds(k * 16, 16)]
            e = plsc.load_gather(asv, [sidx]) + plsc.load_gather(adv, [didx])
            e = jnp.where(e > 0, e, 0.2 * e)
            pbuf[pl.ds(k * 16, 16)] = jnp.exp(e - cvec)
            return c2

        lax.fori_loop(0, CW // 16, vreg_body, 0, unroll=True)
        rcp.wait()

        @plsc.parallel_loop(0, CW, step=16)
        def row_body(g):
            pv = pbuf[pl.ds(g, 16)]
            for l in range(16):
                ps = pv[l]
                for k2 in range(DA // 16):
                    rows[g + l, pl.ds(k2 * 16, 16)] = (
                        rows[g + l, pl.ds(k2 * 16, 16)] * ps)

        pltpu.sync_copy(rows, acc_sh.at[idxv.at[1]], add=True)
        return carry

    lax.fori_loop(0, CH, chunk_body, 0)
    plsc.subcore_barrier()

    sl = pl.ds(si * ACC_PER_TILE, ACC_PER_TILE)
    pltpu.sync_copy(acc_sh.at[sl], acc_out.at[ci, sl])


@functools.partial(
    pl.kernel,
    out_type=jax.ShapeDtypeStruct((2, NACC, DA), jnp.float32),
    mesh=plsc.VectorSubcoreMesh(core_axis_name="c", subcore_axis_name="s"),
    compiler_params=pltpu.CompilerParams(needs_layout_passes=False,
                                         use_tc_tiling_on_sc=False),
    scratch_types=[
        pltpu.VMEM_SHARED((NACC, DA), jnp.float32),
        pltpu.VMEM((NP,), jnp.float32),
        pltpu.VMEM((NP,), jnp.float32),
        pltpu.VMEM((2, CW), jnp.int32),
        pltpu.VMEM((CW,), jnp.float32),
        pltpu.VMEM((CW, DA), jnp.float32),
        pltpu.VMEM((16,), jnp.float32),
        pltpu.SemaphoreType.DMA,
    ],
)
def _sc_edge(*refs):
    _sc_edge_body(*refs)


def _comb_body(acc_ref, skip_ref, b_ref, out_ref):
    full = acc_ref[0, :, :] + acc_ref[1, :, :]
    num = full[:, :D]
    dn = full[:, D:D + 1] + 1e-16
    h = num / dn + skip_ref[...] + b_ref[...][None, :]
    out_ref[...] = jnp.maximum(h, 0.0)


def _comb(acc, skip, b):
    blk = 1016
    return pl.pallas_call(
        _comb_body,
        grid=(NACC // blk,),
        in_specs=[
            pl.BlockSpec((2, blk, DA), lambda i: (0, i, 0)),
            pl.BlockSpec((blk, D), lambda i: (i, 0)),
            pl.BlockSpec((D,), lambda i: (0,)),
        ],
        out_specs=pl.BlockSpec((blk, D), lambda i: (i, 0)),
        out_shape=jax.ShapeDtypeStruct((NACC, D), jnp.float32),
    )(acc, skip, b)


def _layer(x_pad, idx3, zr, w_src, w_dst, att_src, att_dst, b, wl, bl):
    h, skip, a_s, a_d, mas, mad = _mm(x_pad, w_src, wl, bl, w_dst, att_dst, att_src)
    cb = mas[0, 0] + mad[0, 0]
    c = jnp.where(cb > 0, cb, 0.2 * cb)
    cvec = jnp.full((16,), c, jnp.float32)
    acc = _sc_edge(h, a_s, a_d, idx3, cvec, zr)
    out = _comb(acc, skip[:NACC], b)
    return jnp.pad(out, ((0, NP - NACC), (0, 0)))


def kernel(x, edge_index, W1_src, W1_dst, att1_src, att1_dst, b1, Wl1, bl1,
           W2_src, W2_dst, att2_src, att2_dst, b2, Wl2, bl2):
    x_pad = jnp.pad(x, ((0, NP - N), (0, 0)))
    src = edge_index[0].astype(jnp.int32).reshape(NW, EPW)
    dst = edge_index[1].astype(jnp.int32).reshape(NW, EPW)
    pad = ((0, 0), (0, EPP - EPW))
    src3 = jnp.pad(src, pad, constant_values=NP - 1).reshape(NW, CH, CW)
    dst3 = jnp.pad(dst, pad, constant_values=PAD_DST).reshape(NW, CH, CW)
    idx3 = jnp.stack([src3, dst3], axis=2)
    zr = jnp.zeros((ACC_PER_TILE, DA), jnp.float32)

    h = _layer(x_pad, idx3, zr,
               W1_src, W1_dst, att1_src, att1_dst, b1, Wl1, bl1)
    out = _layer(h, idx3, zr,
                 W2_src, W2_dst, att2_src, att2_dst, b2, Wl2, bl2)
    return out[:N]
